# Initial kernel scaffold; baseline (speedup 1.0000x reference)
import jax
import jax.numpy as jnp
from jax.experimental import pallas as pl


def kernel(word_table, ctx_table, words, positive_contexts, negative_contexts):
    raise NotImplementedError("write your pallas kernel here")


# speedup vs baseline: 1.8302x; 1.8302x over previous
import functools

import jax
import jax.numpy as jnp
from jax import lax
from jax.experimental import pallas as pl
from jax.experimental.pallas import tpu as pltpu
from jax.experimental.pallas import tpu_sc as plsc

_NC = 2
_NS = 16
_NW = _NC * _NS
_L = 16
_IDXW = 128
_CH = 1024


@functools.lru_cache(maxsize=None)
def _build(V, D, B, P, N):
  wpw = B // _NW
  pos_pw = wpw * P
  neg_pw = wpw * N
  n_idx = _CH // _IDXW
  widx_rows = wpw // _IDXW
  groups = _CH // _L

  mesh = plsc.VectorSubcoreMesh(
      core_axis_name="c", subcore_axis_name="s",
      num_cores=_NC, num_subcores=_NS)

  @functools.partial(
      pl.kernel,
      out_type=(jax.ShapeDtypeStruct((B * P,), jnp.float32),
                jax.ShapeDtypeStruct((B * N,), jnp.float32)),
      mesh=mesh,
      compiler_params=pltpu.CompilerParams(use_tc_tiling_on_sc=False,
                                           needs_layout_passes=False),
      scratch_types=[
          pltpu.VMEM((widx_rows, _IDXW), jnp.int32),
          pltpu.VMEM((wpw, D), jnp.float32),
          pltpu.VMEM((n_idx, _IDXW), jnp.int32),
          pltpu.VMEM((_CH, D), jnp.float32),
          pltpu.VMEM((_CH,), jnp.float32),
          pltpu.SemaphoreType.DMA,
      ],
  )
  def run(wt_hbm, ct_hbm, words_hbm, pos_hbm, neg_hbm,
          out_p_hbm, out_n_hbm,
          widx_v, wrows_v, cidx_v, crows_v, out_v, sem):
    wid = lax.axis_index("s") * _NC + lax.axis_index("c")
    lane = lax.iota(jnp.int32, _L)

    pltpu.sync_copy(words_hbm.at[wid], widx_v)
    wcps = [
        pltpu.async_copy(wt_hbm.at[widx_v.at[j]],
                         wrows_v.at[pl.ds(j * _IDXW, _IDXW)], sem)
        for j in range(widx_rows)
    ]
    for cp in wcps:
      cp.wait()

    def make_pass(ctx_idx_hbm, out_hbm, per_worker, reps):
      base = wid * per_worker
      n_chunks = per_worker // _CH

      def chunk_body(c, _):
        row0 = pl.multiple_of((base + c * _CH) // _IDXW, 8)
        pltpu.sync_copy(ctx_idx_hbm.at[pl.ds(row0, n_idx)], cidx_v)
        cps = [
            pltpu.async_copy(ct_hbm.at[cidx_v.at[j]],
                             crows_v.at[pl.ds(j * _IDXW, _IDXW)], sem)
            for j in range(n_idx)
        ]
        for cp in cps:
          cp.wait()

        def group_body(g, _):
          t = c * _CH + g * _L + lane
          w = lax.div(t, jnp.int32(reps))
          rows = g * _L + lane
          accs = [jnp.zeros((_L,), jnp.float32) for _ in range(4)]
          for d in range(D):
            dd = jnp.full((_L,), d, jnp.int32)
            wv = plsc.load_gather(wrows_v, [w, dd])
            cv = plsc.load_gather(crows_v, [rows, dd])
            accs[d % 4] = accs[d % 4] + wv * cv
          acc = (accs[0] + accs[1]) + (accs[2] + accs[3])
          out_v[pl.ds(g * _L, _L)] = 1.0 / (1.0 + jnp.exp(-acc))
          return 0

        lax.fori_loop(0, groups, group_body, 0)
        pltpu.sync_copy(out_v, out_hbm.at[pl.ds(base + c * _CH, _CH)])
        return 0

      lax.fori_loop(0, n_chunks, chunk_body, 0)

    make_pass(pos_hbm, out_p_hbm, pos_pw, P)
    make_pass(neg_hbm, out_n_hbm, neg_pw, N)

  return run


def kernel(word_table, ctx_table, words, positive_contexts, negative_contexts):
  V, D = word_table.shape
  B = words.shape[0]
  P = positive_contexts.shape[1]
  N = negative_contexts.shape[1]
  run = _build(V, D, B, P, N)
  words2 = words.astype(jnp.int32).reshape(_NW, (B // _NW) // _IDXW, _IDXW)
  pos2 = positive_contexts.astype(jnp.int32).reshape((B * P) // _IDXW, _IDXW)
  neg2 = negative_contexts.astype(jnp.int32).reshape((B * N) // _IDXW, _IDXW)
  return run(word_table, ctx_table, words2, pos2, neg2)


# speedup vs baseline: 3.4607x; 1.8909x over previous
import functools

import jax
import jax.numpy as jnp
from jax import lax
from jax.experimental import pallas as pl
from jax.experimental.pallas import tpu as pltpu
from jax.experimental.pallas import tpu_sc as plsc

_NC = 2
_NS = 16
_NW = _NC * _NS
_L = 16
_IDXW = 128
_WC_POS = 32
_WC_NEG = 16


@functools.lru_cache(maxsize=None)
def _build(V, D, B, P, N):
  wpw = B // _NW
  cr_pos = _WC_POS * P
  cr_neg = _WC_NEG * N
  cr_max = max(cr_pos, cr_neg)
  wc_max = max(_WC_POS, _WC_NEG)

  mesh = plsc.VectorSubcoreMesh(
      core_axis_name="c", subcore_axis_name="s",
      num_cores=_NC, num_subcores=_NS)

  @functools.partial(
      pl.kernel,
      out_type=(jax.ShapeDtypeStruct((B * P,), jnp.float32),
                jax.ShapeDtypeStruct((B * N,), jnp.float32)),
      mesh=mesh,
      compiler_params=pltpu.CompilerParams(use_tc_tiling_on_sc=False,
                                           needs_layout_passes=False),
      scratch_types=[
          pltpu.VMEM((wpw,), jnp.int32),
          pltpu.VMEM((cr_max,), jnp.int32),
          pltpu.VMEM((cr_max,), jnp.int32),
          pltpu.VMEM((cr_max, D), jnp.float32),
          pltpu.VMEM((cr_max, D), jnp.float32),
          pltpu.VMEM((wc_max, D), jnp.float32),
          pltpu.VMEM((wc_max, D), jnp.float32),
          pltpu.VMEM((cr_max,), jnp.float32),
          pltpu.VMEM((cr_max * 17,), jnp.float32),
          pltpu.SemaphoreType.DMA,
          pltpu.SemaphoreType.DMA,
      ],
  )
  def run(wt_hbm, ct_hbm, words_hbm, pos_hbm, neg_hbm,
          out_p_hbm, out_n_hbm,
          widx_v, cidx0, cidx1, crows0, crows1, wrows0, wrows1,
          out_v, psum_v, sem0, sem1):
    wid = lax.axis_index("s") * _NC + lax.axis_index("c")
    pltpu.sync_copy(words_hbm.at[pl.ds(wid * wpw, wpw)], widx_v)

    def do_pass(ctx_idx_hbm, out_hbm, wc, reps):
      cr = wc * reps
      n_chunks = wpw // wc
      base = wid * wpw * reps
      cidx = (cidx0, cidx1)
      crows = (crows0, crows1)
      wrows = (wrows0, wrows1)
      sems = (sem0, sem1)
      n_dma = cr // _IDXW
      rem = cr - n_dma * _IDXW

      def fire(cc, b):
        pltpu.sync_copy(ctx_idx_hbm.at[pl.ds(base + cc * cr, cr)],
                        cidx[b].at[pl.ds(0, cr)])
        for j in range(n_dma):
          pltpu.async_copy(
              ct_hbm.at[cidx[b].at[pl.ds(j * _IDXW, _IDXW)]],
              crows[b].at[pl.ds(j * _IDXW, _IDXW)], sems[b])
        if rem:
          pltpu.async_copy(
              ct_hbm.at[cidx[b].at[pl.ds(n_dma * _IDXW, rem)]],
              crows[b].at[pl.ds(n_dma * _IDXW, rem)], sems[b])
        pltpu.async_copy(wt_hbm.at[widx_v.at[pl.ds(cc * wc, wc)]],
                         wrows[b].at[pl.ds(0, wc)], sems[b])

      def drain(b):
        for j in range(n_dma):
          pltpu.make_async_copy(
              ct_hbm.at[cidx[b].at[pl.ds(j * _IDXW, _IDXW)]],
              crows[b].at[pl.ds(j * _IDXW, _IDXW)], sems[b]).wait()
        if rem:
          pltpu.make_async_copy(
              ct_hbm.at[cidx[b].at[pl.ds(n_dma * _IDXW, rem)]],
              crows[b].at[pl.ds(n_dma * _IDXW, rem)], sems[b]).wait()
        pltpu.make_async_copy(wt_hbm.at[widx_v.at[pl.ds(0, wc)]],
                              wrows[b].at[pl.ds(0, wc)], sems[b]).wait()

      def compute(c, b):
        def word_body(iw, _):
          wr = [wrows[b][iw, pl.ds(k * _L, _L)] for k in range(D // _L)]
          for j in range(reps):
            r = iw * reps + j
            acc = None
            for k in range(D // _L):
              term = wr[k] * crows[b][r, pl.ds(k * _L, _L)]
              acc = term if acc is None else acc + term
            psum_v[pl.ds(r * 17, _L)] = acc
          return 0

        lax.fori_loop(0, wc, word_body, 0)

        lane17 = lax.iota(jnp.int32, _L) * 17

        def red_body(g, _):
          va = lane17 + g * (17 * _L)
          s = None
          for d in range(_L):
            col = plsc.load_gather(psum_v, [va + d])
            s = col if s is None else s + col
          out_v[pl.ds(g * _L, _L)] = 1.0 / (1.0 + jnp.exp(-s))
          return 0

        lax.fori_loop(0, cr // _L, red_body, 0)
        pltpu.sync_copy(out_v.at[pl.ds(0, cr)],
                        out_hbm.at[pl.ds(base + c * cr, cr)])

      fire(0, 0)
      fire(1, 1)

      @pl.loop(0, n_chunks, step=2)
      def _(c):
        for b in range(2):
          drain(b)
          compute(c + b, b)
          fire(jnp.minimum(c + 2 + b, n_chunks - 1), b)

      drain(0)
      drain(1)

    do_pass(pos_hbm, out_p_hbm, _WC_POS, P)
    do_pass(neg_hbm, out_n_hbm, _WC_NEG, N)

  return run


def kernel(word_table, ctx_table, words, positive_contexts, negative_contexts):
  V, D = word_table.shape
  B = words.shape[0]
  P = positive_contexts.shape[1]
  N = negative_contexts.shape[1]
  run = _build(V, D, B, P, N)
  return run(word_table, ctx_table,
             words.astype(jnp.int32),
             positive_contexts.astype(jnp.int32).reshape(B * P),
             negative_contexts.astype(jnp.int32).reshape(B * N))


# speedup vs baseline: 3.5225x; 1.0179x over previous
import functools

import jax
import jax.numpy as jnp
from jax import lax
from jax.experimental import pallas as pl
from jax.experimental.pallas import tpu as pltpu
from jax.experimental.pallas import tpu_sc as plsc

_NC = 2
_NS = 16
_NW = _NC * _NS
_L = 16
_IDXW = 128
_WC_POS = 16
_WC_NEG = 8


@functools.lru_cache(maxsize=None)
def _build(V, D, B, P, N):
  wpw = B // _NW
  cr_pos = _WC_POS * P
  cr_neg = _WC_NEG * N
  cr_max = max(cr_pos, cr_neg)
  idx_max = wpw * max(P, N)

  mesh = plsc.VectorSubcoreMesh(
      core_axis_name="c", subcore_axis_name="s",
      num_cores=_NC, num_subcores=_NS)

  @functools.partial(
      pl.kernel,
      out_type=(jax.ShapeDtypeStruct((B * P,), jnp.float32),
                jax.ShapeDtypeStruct((B * N,), jnp.float32)),
      mesh=mesh,
      compiler_params=pltpu.CompilerParams(use_tc_tiling_on_sc=False,
                                           needs_layout_passes=False),
      scratch_types=[
          pltpu.VMEM((wpw,), jnp.int32),
          pltpu.VMEM((wpw, D), jnp.float32),
          pltpu.VMEM((idx_max,), jnp.int32),
          pltpu.VMEM((cr_max, D), jnp.float32),
          pltpu.VMEM((cr_max, D), jnp.float32),
          pltpu.VMEM((cr_max,), jnp.float32),
          pltpu.VMEM((cr_max,), jnp.float32),
          pltpu.VMEM((cr_max * 17,), jnp.float32),
          pltpu.SemaphoreType.DMA,
          pltpu.SemaphoreType.DMA,
          pltpu.SemaphoreType.DMA,
          pltpu.SemaphoreType.DMA,
      ],
  )
  def run(wt_hbm, ct_hbm, words_hbm, pos_hbm, neg_hbm,
          out_p_hbm, out_n_hbm,
          widx_v, wrows_v, cidx_v, crows0, crows1, out0, out1,
          psum_v, semg0, semg1, semo0, semo1):
    wid = lax.axis_index("s") * _NC + lax.axis_index("c")
    lane17 = lax.iota(jnp.int32, _L) * 17

    pltpu.sync_copy(words_hbm.at[pl.ds(wid * wpw, wpw)], widx_v)
    wcps = [
        pltpu.async_copy(wt_hbm.at[widx_v.at[pl.ds(j * _IDXW, _IDXW)]],
                         wrows_v.at[pl.ds(j * _IDXW, _IDXW)], semg0)
        for j in range(wpw // _IDXW)
    ]
    for cp in wcps:
      cp.wait()

    def do_pass(ctx_idx_hbm, out_hbm, wc, reps):
      cr = wc * reps
      n_chunks = wpw // wc
      base = wid * wpw * reps
      crows = (crows0, crows1)
      outs = (out0, out1)
      semg = (semg0, semg1)
      semo = (semo0, semo1)
      n_full = cr // _IDXW
      rem = cr - n_full * _IDXW
      dmas = [(j * _IDXW, _IDXW) for j in range(n_full)]
      if rem:
        dmas.append((n_full * _IDXW, rem))

      pltpu.sync_copy(ctx_idx_hbm.at[pl.ds(base, wpw * reps)],
                      cidx_v.at[pl.ds(0, wpw * reps)])

      def fire(cc, b):
        for (o, n) in dmas:
          pltpu.async_copy(
              ct_hbm.at[cidx_v.at[pl.ds(cc * cr + o, n)]],
              crows[b].at[pl.ds(o, n)], semg[b])

      def drain(b):
        for (o, n) in dmas:
          pltpu.make_async_copy(
              ct_hbm.at[cidx_v.at[pl.ds(o, n)]],
              crows[b].at[pl.ds(o, n)], semg[b]).wait()

      def compute(c, b):
        cstart = c * wc

        def word_body(iw, _):
          wr = [wrows_v[cstart + iw, pl.ds(k * _L, _L)]
                for k in range(D // _L)]
          for j in range(reps):
            r = iw * reps + j
            acc = None
            for k in range(D // _L):
              term = wr[k] * crows[b][r, pl.ds(k * _L, _L)]
              acc = term if acc is None else acc + term
            psum_v[pl.ds(r * 17, _L)] = acc
          return 0

        lax.fori_loop(0, wc, word_body, 0)

        def red_body(g, _):
          va = lane17 + g * (17 * _L)
          s = None
          for d in range(_L):
            col = plsc.load_gather(psum_v, [va + d])
            s = col if s is None else s + col
          outs[b][pl.ds(g * _L, _L)] = 1.0 / (1.0 + jnp.exp(-s))
          return 0

        lax.fori_loop(0, cr // _L, red_body, 0)

      def out_wait(b):
        pltpu.make_async_copy(outs[b].at[pl.ds(0, cr)],
                              out_hbm.at[pl.ds(base, cr)], semo[b]).wait()

      fire(0, 0)
      fire(1, 1)
      pltpu.async_copy(out_hbm.at[pl.ds(base, cr)], outs[0].at[pl.ds(0, cr)],
                       semo[0])
      pltpu.async_copy(out_hbm.at[pl.ds(base, cr)], outs[1].at[pl.ds(0, cr)],
                       semo[1])

      @pl.loop(0, n_chunks, step=2)
      def _(c):
        for b in range(2):
          drain(b)
          out_wait(b)
          compute(c + b, b)
          pltpu.async_copy(outs[b].at[pl.ds(0, cr)],
                           out_hbm.at[pl.ds(base + (c + b) * cr, cr)],
                           semo[b])
          fire(jnp.minimum(c + 2 + b, n_chunks - 1), b)

      drain(0)
      drain(1)
      out_wait(0)
      out_wait(1)

    do_pass(pos_hbm, out_p_hbm, _WC_POS, P)
    do_pass(neg_hbm, out_n_hbm, _WC_NEG, N)

  return run


def kernel(word_table, ctx_table, words, positive_contexts, negative_contexts):
  V, D = word_table.shape
  B = words.shape[0]
  P = positive_contexts.shape[1]
  N = negative_contexts.shape[1]
  run = _build(V, D, B, P, N)
  return run(word_table, ctx_table,
             words.astype(jnp.int32),
             positive_contexts.astype(jnp.int32).reshape(B * P),
             negative_contexts.astype(jnp.int32).reshape(B * N))


# speedup vs baseline: 3.5634x; 1.0116x over previous
import functools

import jax
import jax.numpy as jnp
from jax import lax
from jax.experimental import pallas as pl
from jax.experimental.pallas import tpu as pltpu
from jax.experimental.pallas import tpu_sc as plsc

_NC = 2
_NS = 16
_NW = _NC * _NS
_L = 16
_IDXW = 128
_WC_POS = 16
_WC_NEG = 8


@functools.lru_cache(maxsize=None)
def _build(V, D, B, P, N):
  wpw = B // _NW
  cr_pos = _WC_POS * P
  cr_neg = _WC_NEG * N
  cr_max = max(cr_pos, cr_neg)
  idx_max = wpw * max(P, N)

  mesh = plsc.VectorSubcoreMesh(
      core_axis_name="c", subcore_axis_name="s",
      num_cores=_NC, num_subcores=_NS)

  @functools.partial(
      pl.kernel,
      out_type=(jax.ShapeDtypeStruct((B * P,), jnp.float32),
                jax.ShapeDtypeStruct((B * N,), jnp.float32)),
      mesh=mesh,
      compiler_params=pltpu.CompilerParams(use_tc_tiling_on_sc=False,
                                           needs_layout_passes=False),
      scratch_types=[
          pltpu.VMEM((wpw,), jnp.int32),
          pltpu.VMEM((wpw, D), jnp.float32),
          pltpu.VMEM((idx_max,), jnp.int32),
          pltpu.VMEM((cr_max, D), jnp.float32),
          pltpu.VMEM((cr_max, D), jnp.float32),
          pltpu.VMEM((cr_max,), jnp.float32),
          pltpu.VMEM((cr_max,), jnp.float32),
          pltpu.VMEM((cr_max * 17,), jnp.float32),
          pltpu.SemaphoreType.DMA,
          pltpu.SemaphoreType.DMA,
          pltpu.SemaphoreType.DMA,
          pltpu.SemaphoreType.DMA,
      ],
  )
  def run(wt_hbm, ct_hbm, words_hbm, pos_hbm, neg_hbm,
          out_p_hbm, out_n_hbm,
          widx_v, wrows_v, cidx_v, crows0, crows1, out0, out1,
          psum_v, semg0, semg1, semo0, semo1):
    wid = lax.axis_index("s") * _NC + lax.axis_index("c")
    lane17 = lax.iota(jnp.int32, _L) * 17

    pltpu.sync_copy(words_hbm.at[pl.ds(wid * wpw, wpw)], widx_v)
    wcps = [
        pltpu.async_copy(wt_hbm.at[widx_v.at[pl.ds(j * _IDXW, _IDXW)]],
                         wrows_v.at[pl.ds(j * _IDXW, _IDXW)], semg0)
        for j in range(wpw // _IDXW)
    ]
    for cp in wcps:
      cp.wait()

    def do_pass(ctx_idx_hbm, out_hbm, wc, reps):
      cr = wc * reps
      n_chunks = wpw // wc
      base = wid * wpw * reps
      crows = (crows0, crows1)
      outs = (out0, out1)
      semg = (semg0, semg1)
      semo = (semo0, semo1)
      n_full = cr // _IDXW
      rem = cr - n_full * _IDXW
      dmas = [(j * _IDXW, _IDXW) for j in range(n_full)]
      if rem:
        dmas.append((n_full * _IDXW, rem))

      pltpu.sync_copy(ctx_idx_hbm.at[pl.ds(base, wpw * reps)],
                      cidx_v.at[pl.ds(0, wpw * reps)])

      def fire(cc, b):
        for (o, n) in dmas:
          pltpu.async_copy(
              ct_hbm.at[cidx_v.at[pl.ds(cc * cr + o, n)]],
              crows[b].at[pl.ds(o, n)], semg[b])

      def drain(b):
        for (o, n) in dmas:
          pltpu.make_async_copy(
              ct_hbm.at[cidx_v.at[pl.ds(o, n)]],
              crows[b].at[pl.ds(o, n)], semg[b]).wait()

      def compute(c, b):
        cstart = c * wc

        def word_body(iw, _):
          wr = [wrows_v[cstart + iw, pl.ds(k * _L, _L)]
                for k in range(D // _L)]
          for j in range(reps):
            r = iw * reps + j
            ts = [wr[k] * crows[b][r, pl.ds(k * _L, _L)]
                  for k in range(D // _L)]
            while len(ts) > 1:
              ts = [ts[i] + ts[i + 1] for i in range(0, len(ts) - 1, 2)] + (
                  [ts[-1]] if len(ts) % 2 else [])
            psum_v[pl.ds(r * 17, _L)] = ts[0]
          return 0

        lax.fori_loop(0, wc, word_body, 0)

        def red_body(g, _):
          va = lane17 + g * (17 * _L)
          cols = [plsc.load_gather(psum_v, [va + d]) for d in range(_L)]
          while len(cols) > 1:
            cols = [cols[i] + cols[i + 1] for i in range(0, len(cols), 2)]
          outs[b][pl.ds(g * _L, _L)] = 1.0 / (1.0 + jnp.exp(-cols[0]))
          return 0

        lax.fori_loop(0, cr // _L, red_body, 0)

      def out_wait(b):
        pltpu.make_async_copy(outs[b].at[pl.ds(0, cr)],
                              out_hbm.at[pl.ds(base, cr)], semo[b]).wait()

      fire(0, 0)
      fire(1, 1)
      pltpu.async_copy(out_hbm.at[pl.ds(base, cr)], outs[0].at[pl.ds(0, cr)],
                       semo[0])
      pltpu.async_copy(out_hbm.at[pl.ds(base, cr)], outs[1].at[pl.ds(0, cr)],
                       semo[1])

      @pl.loop(0, n_chunks, step=2)
      def _(c):
        for b in range(2):
          drain(b)
          out_wait(b)
          compute(c + b, b)
          pltpu.async_copy(outs[b].at[pl.ds(0, cr)],
                           out_hbm.at[pl.ds(base + (c + b) * cr, cr)],
                           semo[b])
          fire(jnp.minimum(c + 2 + b, n_chunks - 1), b)

      drain(0)
      drain(1)
      out_wait(0)
      out_wait(1)

    do_pass(pos_hbm, out_p_hbm, _WC_POS, P)
    do_pass(neg_hbm, out_n_hbm, _WC_NEG, N)

  return run


def kernel(word_table, ctx_table, words, positive_contexts, negative_contexts):
  V, D = word_table.shape
  B = words.shape[0]
  P = positive_contexts.shape[1]
  N = negative_contexts.shape[1]
  run = _build(V, D, B, P, N)
  return run(word_table, ctx_table,
             words.astype(jnp.int32),
             positive_contexts.astype(jnp.int32).reshape(B * P),
             negative_contexts.astype(jnp.int32).reshape(B * N))
